# baseline (speedup 1.0000x reference)
import functools

import jax
import jax.numpy as jnp
from jax import lax
from jax.experimental import pallas as pl
from jax.experimental.pallas import tpu as pltpu
from jax.experimental.pallas import tpu_sc as plsc

NUM_ROWS = 16384 * 26
FEATURES = 64

NC = 2
NS = 16
NW = NC * NS
ROWS_PER_W = NUM_ROWS // NW
CHUNK = 512
N_CHUNKS = ROWS_PER_W // CHUNK


def _gather_body(idx_hbm, table_hbm, out_hbm,
                 idx_v, rows0, rows1, gsem0, gsem1, osem0, osem1):
    wid = lax.axis_index("s") * NC + lax.axis_index("c")
    base = wid * ROWS_PER_W

    pltpu.sync_copy(idx_hbm.at[pl.ds(base, ROWS_PER_W)], idx_v)

    rows = (rows0, rows1)
    gsems = (gsem0, gsem1)
    osems = (osem0, osem1)

    gather_h = [None, None]
    out_h = [None, None]
    for j in range(N_CHUNKS + 1):
        slot = j % 2
        if j < N_CHUNKS:
            if out_h[slot] is not None:
                out_h[slot].wait()
                out_h[slot] = None
            gather_h[slot] = pltpu.async_copy(
                table_hbm.at[idx_v.at[pl.ds(j * CHUNK, CHUNK)]],
                rows[slot], gsems[slot])
        if j >= 1:
            prev = (j - 1) % 2
            gather_h[prev].wait()
            out_h[prev] = pltpu.async_copy(
                rows[prev],
                out_hbm.at[pl.ds(base + (j - 1) * CHUNK, CHUNK)],
                osems[prev])
    for h in out_h:
        if h is not None:
            h.wait()


@functools.partial(jax.jit, static_argnames=())
def _run(idx_flat, embedding):
    mesh = plsc.VectorSubcoreMesh(core_axis_name="c", subcore_axis_name="s")
    k = functools.partial(
        pl.kernel,
        mesh=mesh,
        compiler_params=pltpu.CompilerParams(use_tc_tiling_on_sc=False),
        out_type=jax.ShapeDtypeStruct((NUM_ROWS, FEATURES), jnp.float32),
        scratch_types=[
            pltpu.VMEM((ROWS_PER_W,), jnp.int32),
            pltpu.VMEM((CHUNK, FEATURES), jnp.float32),
            pltpu.VMEM((CHUNK, FEATURES), jnp.float32),
            pltpu.SemaphoreType.DMA,
            pltpu.SemaphoreType.DMA,
            pltpu.SemaphoreType.DMA,
            pltpu.SemaphoreType.DMA,
        ],
    )(_gather_body)
    return k(idx_flat, embedding)


def kernel(inputs, embedding):
    idx = inputs.reshape(NUM_ROWS).astype(jnp.int32)
    out = _run(idx, embedding)
    return out.reshape(inputs.shape + (FEATURES,))
